# baseline (speedup 1.0000x reference)
import jax
import jax.numpy as jnp
from jax import lax
from jax.experimental import pallas as pl
from jax.experimental.pallas import tpu as pltpu
from jax.experimental.pallas import tpu_sc as plsc

_N = 2048
_D = 256
_K = 33
_KSEL = _K + 1
_MAX_ITER = 3
_ETA = 0.01
_EPS = 1e-8
_BR = 256
_NB = _N // _BR


def _prep_kernel(x_ref, xn_ref, d2_ref):
    x = x_ref[...]
    mean = jnp.mean(x, axis=0, keepdims=True)
    xc = x - mean
    var = jnp.sum(xc * xc, axis=0, keepdims=True) * (1.0 / (_N - 1))
    std = jnp.sqrt(var)
    xn = xc / (std + _EPS)
    xn_ref[...] = xn

    sq_all = jnp.sum(xn * xn, axis=1)[None, :]
    xnb = xn.astype(jnp.bfloat16)

    def body(rb, _):
        xr = xn_ref[pl.ds(rb * _BR, _BR), :]
        g = lax.dot_general(xr.astype(jnp.bfloat16), xnb,
                            (((1,), (1,)), ((), ())),
                            preferred_element_type=jnp.float32)
        sqr = jnp.sum(xr * xr, axis=1, keepdims=True)
        d2 = jnp.maximum(sqr + sq_all - 2.0 * g, 0.0)
        cols = lax.broadcasted_iota(jnp.int32, (_BR, _N), 1)
        rows = rb * _BR + lax.broadcasted_iota(jnp.int32, (_BR, _N), 0)
        d2 = jnp.where(cols == rows, jnp.float32(jnp.inf), d2)
        d2_ref[pl.ds(rb * _BR, _BR), :] = d2
        return 0

    lax.fori_loop(0, _NB, body, 0)


def _thresh_kernel(d2_ref, thr_ref):
    def body(rb, _):
        bits = lax.bitcast_convert_type(
            d2_ref[pl.ds(rb * _BR, _BR), :], jnp.int32)
        lo = jnp.zeros((_BR, 1), jnp.int32)
        hi = jnp.full((_BR, 1), jnp.int32(2**31 - 1))

        def bs(i, carry):
            lo, hi = carry
            mid = lo + lax.shift_right_logical(hi - lo, 1)
            cnt = jnp.sum((bits <= mid).astype(jnp.int32), axis=1,
                          keepdims=True)
            take = cnt >= _KSEL
            hi = jnp.where(take, mid, hi)
            lo = jnp.where(take, lo, mid + 1)
            return lo, hi

        lo, hi = lax.fori_loop(0, 31, bs, (lo, hi))
        thr_ref[pl.ds(rb * _BR, _BR), :] = lax.bitcast_convert_type(
            hi, jnp.float32)
        return 0

    lax.fori_loop(0, _NB, body, 0)


_SC_LANES = 16
_SC_TILES = 32
_ROWS_PER_TILE = _N // _SC_TILES
_VPR = _N // _SC_LANES
_MAXF_BITS = 2**31 - 1
_M_KEEP = 8


def _sc_row_threshold(row_ref):
    big = jnp.full((_SC_LANES,), 3.4028235e38, jnp.float32)

    def step_a(i, ms):
        v = row_ref[pl.ds(i * _SC_LANES, _SC_LANES)]
        new = []
        cur = v
        for m in ms:
            new.append(jnp.minimum(m, cur))
            cur = jnp.maximum(m, cur)
        return tuple(new)

    ms = lax.fori_loop(0, _VPR, step_a, (big,) * _M_KEEP, unroll=8)

    def popc(mask):
        return plsc.all_reduce_population_count(mask)

    zero = jnp.zeros((_SC_LANES,), jnp.int32)
    top = jnp.full((_SC_LANES,), _MAXF_BITS, jnp.int32)

    def merge2(x, y):
        ry = jnp.flip(y, axis=0)
        return jnp.sort(jnp.minimum(x, ry)), jnp.sort(jnp.maximum(x, ry))

    def drop_top(x, y):
        return jnp.sort(jnp.minimum(x, jnp.flip(y, axis=0)))

    def low48(p0, p1, q0, q1):
        c2 = drop_top(p1, q1)
        e0, e1 = merge2(p0, q0)
        f0, f1 = merge2(e1, c2)
        g0, g1 = merge2(e0, f0)
        return g0, g1, f1

    s = [jnp.sort(m) for m in ms]
    a0, a1 = merge2(s[0], s[1])
    b0, b1 = merge2(s[2], s[3])
    c0, c1 = merge2(s[4], s[5])
    d0, d1 = merge2(s[6], s[7])
    x0, x1, x2 = low48(a0, a1, b0, b1)
    y0, y1, y2 = low48(c0, c1, d0, d1)
    cc = drop_top(x2, y2)
    dd = drop_top(x1, y1)
    ee = drop_top(dd, cc)
    f0, f1 = merge2(x0, y0)
    g0, g1 = merge2(f1, ee)
    h0, h1 = merge2(f0, g0)
    del h0, h1
    t0 = jnp.min(g1)

    def step_c(i, c):
        v = row_ref[pl.ds(i * _SC_LANES, _SC_LANES)]
        return c + popc(v <= t0)

    overflow = jnp.max(popc(ms[_M_KEEP - 1] <= t0))

    def full_bisect(_):
        def step(i, carry):
            lo, hi = carry
            mid = lo + lax.shift_right_logical(hi - lo, 1)

            def cnt_step(j, cc2):
                bits = lax.bitcast_convert_type(
                    row_ref[pl.ds(j * _SC_LANES, _SC_LANES)], jnp.int32)
                return cc2 + popc(bits <= mid)

            cnt = lax.fori_loop(0, _VPR, cnt_step, zero, unroll=8)
            take = cnt >= _K
            return jnp.where(take, lo, mid + 1), jnp.where(take, mid, hi)

        _, hi = lax.fori_loop(0, 31, step, (zero, top))
        return jnp.max(lax.bitcast_convert_type(hi, jnp.float32))

    def verify(_):
        c = lax.fori_loop(0, _VPR, step_c, zero, unroll=8)
        return lax.cond(jnp.max(c) == _K, lambda __: t0, full_bisect, 0)

    return lax.cond(overflow == 0, lambda _: t0, verify, 0)


def _sc_store_lane(out_scr, r, thr_vec):
    blk = lax.div(r, _SC_LANES) * _SC_LANES
    lane = lax.rem(r, _SC_LANES)
    old = out_scr[pl.ds(blk, _SC_LANES)]
    idx = lax.broadcasted_iota(jnp.int32, (_SC_LANES,), 0)
    out_scr[pl.ds(blk, _SC_LANES)] = jnp.where(idx == lane, thr_vec, old)


def _sc_thresh_body(d2_hbm, thr_hbm, buf0, buf1, out_scr, sem0, sem1):
    wid = lax.axis_index("s") * 2 + lax.axis_index("c")
    base = wid * _ROWS_PER_TILE
    last = base + _ROWS_PER_TILE - 1

    pltpu.async_copy(d2_hbm.at[base], buf0, sem0)
    pltpu.async_copy(d2_hbm.at[base + 1], buf1, sem1)

    def body(j, _):
        r0 = base + 2 * j
        pltpu.make_async_copy(d2_hbm.at[r0], buf0, sem0).wait()
        thr0 = _sc_row_threshold(buf0)

        @pl.when(r0 + 2 <= last)
        def _():
            pltpu.async_copy(d2_hbm.at[r0 + 2], buf0, sem0)

        _sc_store_lane(out_scr, 2 * j, thr0)

        pltpu.make_async_copy(d2_hbm.at[r0 + 1], buf1, sem1).wait()
        thr1 = _sc_row_threshold(buf1)

        @pl.when(r0 + 3 <= last)
        def _():
            pltpu.async_copy(d2_hbm.at[r0 + 3], buf1, sem1)

        _sc_store_lane(out_scr, 2 * j + 1, thr1)
        return 0

    lax.fori_loop(0, _ROWS_PER_TILE // 2, body, 0)
    pltpu.sync_copy(out_scr, thr_hbm.at[pl.ds(base, _ROWS_PER_TILE)])


def _sc_thresh(d2):
    mesh = plsc.VectorSubcoreMesh(core_axis_name="c", subcore_axis_name="s")
    thr = pl.kernel(
        _sc_thresh_body,
        out_type=jax.ShapeDtypeStruct((_N,), jnp.float32),
        mesh=mesh,
        compiler_params=pltpu.CompilerParams(needs_layout_passes=False),
        scratch_types=[
            pltpu.VMEM((_N,), jnp.float32),
            pltpu.VMEM((_N,), jnp.float32),
            pltpu.VMEM((_ROWS_PER_TILE,), jnp.float32),
            pltpu.SemaphoreType.DMA,
            pltpu.SemaphoreType.DMA,
        ],
    )(d2)
    return thr


def _exp0_kernel(d2_ref, alpha_ref, e0_ref):
    alpha = alpha_ref[0, 0]

    def body(rb, _):
        d2r = d2_ref[pl.ds(rb * _BR, _BR), :]
        rm = jnp.min(d2r, axis=1, keepdims=True)
        e0_ref[pl.ds(rb * _BR, _BR), :] = jnp.exp(
            alpha * (rm - d2r)).astype(jnp.bfloat16)
        return 0

    lax.fori_loop(0, _NB, body, 0)


def _iter_kernel(xn_ref, d2_ref, thr_ref, e0_ref, alpha_ref, y_out_ref,
                 y_scr, ynext_scr, sq_scr, yb_scr):
    alpha = alpha_ref[0, 0]
    y_scr[...] = xn_ref[...]

    for _t in range(_MAX_ITER):
        def sq_body(rb, _):
            yr = y_scr[pl.ds(rb * _BR, _BR), :]
            yb_scr[pl.ds(rb * _BR, _BR), :] = yr.astype(jnp.bfloat16)
            if _t > 0:
                sq_scr[0, pl.ds(rb * _BR, _BR)] = jnp.sum(yr * yr, axis=1)
            return 0

        lax.fori_loop(0, _NB, sq_body, 0)

        first = _t == 0

        def blk_body(rb, _):
            yr = y_scr[pl.ds(rb * _BR, _BR), :]
            d2r = d2_ref[pl.ds(rb * _BR, _BR), :]
            thr = thr_ref[pl.ds(rb * _BR, _BR), :]
            mask = d2r <= thr
            if first:
                e0r = e0_ref[pl.ds(rb * _BR, _BR), :]
                e = jnp.where(mask, e0r, jnp.bfloat16(0.0))
            else:
                yb = yb_scr[...]
                yrb = yb_scr[pl.ds(rb * _BR, _BR), :]
                g = lax.dot_general(yrb, yb, (((1,), (1,)), ((), ())),
                                    preferred_element_type=jnp.float32)
                sq_row = sq_scr[0, :][None, :]
                logits = alpha * (2.0 * g - sq_row)
                ml = jnp.where(mask, logits, -1e30)
                m = jnp.max(ml, axis=1, keepdims=True)
                e = jnp.where(mask, jnp.exp(ml - m), 0.0).astype(
                    jnp.bfloat16)
            s = jnp.sum(e.astype(jnp.float32), axis=1, keepdims=True)
            yfb = yb_scr[...]
            wy = lax.dot_general(e, yfb, (((1,), (0,)), ((), ())),
                                 preferred_element_type=jnp.float32)
            ynext_scr[pl.ds(rb * _BR, _BR), :] = yr + _ETA * (wy / s - yr)
            return 0

        lax.fori_loop(0, _NB, blk_body, 0)
        y_scr[...] = ynext_scr[...]

    y_out_ref[...] = y_scr[...]


def kernel(x, alpha):
    xn, d2 = pl.pallas_call(
        _prep_kernel,
        out_shape=[
            jax.ShapeDtypeStruct((_N, _D), jnp.float32),
            jax.ShapeDtypeStruct((_N, _N), jnp.float32),
        ],
    )(x)

    thr = _sc_thresh(d2).reshape(_N, 1)

    alpha2d = jnp.asarray(alpha, jnp.float32).reshape(1, 1)
    e0 = pl.pallas_call(
        _exp0_kernel,
        in_specs=[
            pl.BlockSpec(memory_space=pltpu.VMEM),
            pl.BlockSpec(memory_space=pltpu.SMEM),
        ],
        out_shape=jax.ShapeDtypeStruct((_N, _N), jnp.bfloat16),
    )(d2, alpha2d)

    y = pl.pallas_call(
        _iter_kernel,
        in_specs=[
            pl.BlockSpec(memory_space=pltpu.VMEM),
            pl.BlockSpec(memory_space=pltpu.VMEM),
            pl.BlockSpec(memory_space=pltpu.VMEM),
            pl.BlockSpec(memory_space=pltpu.VMEM),
            pl.BlockSpec(memory_space=pltpu.SMEM),
        ],
        out_shape=jax.ShapeDtypeStruct((_N, _D), jnp.float32),
        scratch_shapes=[
            pltpu.VMEM((_N, _D), jnp.float32),
            pltpu.VMEM((_N, _D), jnp.float32),
            pltpu.VMEM((1, _N), jnp.float32),
            pltpu.VMEM((_N, _D), jnp.bfloat16),
        ],
    )(xn, d2, thr, e0, alpha2d)
    return y


# speedup vs baseline: 1.0176x; 1.0176x over previous
import jax
import jax.numpy as jnp
from jax import lax
from jax.experimental import pallas as pl
from jax.experimental.pallas import tpu as pltpu
from jax.experimental.pallas import tpu_sc as plsc

_N = 2048
_D = 256
_K = 33
_MAX_ITER = 3
_ETA = 0.01
_EPS = 1e-8
_BR = 256
_NB = _N // _BR


def _prep_kernel(x_ref, xn_ref, d2_ref):
    x = x_ref[...]
    mean = jnp.mean(x, axis=0, keepdims=True)
    xc = x - mean
    var = jnp.sum(xc * xc, axis=0, keepdims=True) * (1.0 / (_N - 1))
    std = jnp.sqrt(var)
    xn = xc / (std + _EPS)
    xn_ref[...] = xn

    sq_all = jnp.sum(xn * xn, axis=1)[None, :]
    xnb = xn.astype(jnp.bfloat16)

    def body(rb, _):
        xr = xn_ref[pl.ds(rb * _BR, _BR), :]
        g = lax.dot_general(xr.astype(jnp.bfloat16), xnb,
                            (((1,), (1,)), ((), ())),
                            preferred_element_type=jnp.float32)
        sqr = jnp.sum(xr * xr, axis=1, keepdims=True)
        d2 = jnp.maximum(sqr + sq_all - 2.0 * g, 0.0)
        cols = lax.broadcasted_iota(jnp.int32, (_BR, _N), 1)
        rows = rb * _BR + lax.broadcasted_iota(jnp.int32, (_BR, _N), 0)
        d2 = jnp.where(cols == rows, jnp.float32(jnp.inf), d2)
        d2_ref[pl.ds(rb * _BR, _BR), :] = d2
        return 0

    lax.fori_loop(0, _NB, body, 0)


_SC_LANES = 16
_SC_TILES = 32
_ROWS_PER_TILE = _N // _SC_TILES
_VPR = _N // _SC_LANES
_MAXF_BITS = 2**31 - 1
_M_KEEP = 8


def _sc_row_threshold(row_ref):
    big = jnp.full((_SC_LANES,), 3.4028235e38, jnp.float32)

    def step_a(i, ms):
        v = row_ref[pl.ds(i * _SC_LANES, _SC_LANES)]
        new = []
        cur = v
        for m in ms:
            new.append(jnp.minimum(m, cur))
            cur = jnp.maximum(m, cur)
        return tuple(new)

    ms = lax.fori_loop(0, _VPR, step_a, (big,) * _M_KEEP, unroll=8)

    def popc(mask):
        return plsc.all_reduce_population_count(mask)

    zero = jnp.zeros((_SC_LANES,), jnp.int32)
    top = jnp.full((_SC_LANES,), _MAXF_BITS, jnp.int32)

    def merge2(x, y):
        ry = jnp.flip(y, axis=0)
        return jnp.sort(jnp.minimum(x, ry)), jnp.sort(jnp.maximum(x, ry))

    def drop_top(x, y):
        return jnp.sort(jnp.minimum(x, jnp.flip(y, axis=0)))

    def low48(p0, p1, q0, q1):
        c2 = drop_top(p1, q1)
        e0, e1 = merge2(p0, q0)
        f0, f1 = merge2(e1, c2)
        g0, g1 = merge2(e0, f0)
        return g0, g1, f1

    s = [jnp.sort(m) for m in ms]
    a0, a1 = merge2(s[0], s[1])
    b0, b1 = merge2(s[2], s[3])
    c0, c1 = merge2(s[4], s[5])
    d0, d1 = merge2(s[6], s[7])
    x0, x1, x2 = low48(a0, a1, b0, b1)
    y0, y1, y2 = low48(c0, c1, d0, d1)
    cc = drop_top(x2, y2)
    dd = drop_top(x1, y1)
    ee = drop_top(dd, cc)
    f0, f1 = merge2(x0, y0)
    g0, g1 = merge2(f1, ee)
    h0, h1 = merge2(f0, g0)
    del h0, h1
    t0 = jnp.min(g1)

    def step_c(i, c):
        v = row_ref[pl.ds(i * _SC_LANES, _SC_LANES)]
        return c + popc(v <= t0)

    overflow = jnp.max(popc(ms[_M_KEEP - 1] <= t0))

    def full_bisect(_):
        def step(i, carry):
            lo, hi = carry
            mid = lo + lax.shift_right_logical(hi - lo, 1)

            def cnt_step(j, cc2):
                bits = lax.bitcast_convert_type(
                    row_ref[pl.ds(j * _SC_LANES, _SC_LANES)], jnp.int32)
                return cc2 + popc(bits <= mid)

            cnt = lax.fori_loop(0, _VPR, cnt_step, zero, unroll=8)
            take = cnt >= _K
            return jnp.where(take, lo, mid + 1), jnp.where(take, mid, hi)

        _, hi = lax.fori_loop(0, 31, step, (zero, top))
        return jnp.max(lax.bitcast_convert_type(hi, jnp.float32))

    def verify(_):
        c = lax.fori_loop(0, _VPR, step_c, zero, unroll=8)
        return lax.cond(jnp.max(c) == _K, lambda __: t0, full_bisect, 0)

    return lax.cond(overflow == 0, lambda _: t0, verify, 0)


def _sc_store_lane(out_scr, r, thr_vec):
    blk = lax.div(r, _SC_LANES) * _SC_LANES
    lane = lax.rem(r, _SC_LANES)
    old = out_scr[pl.ds(blk, _SC_LANES)]
    idx = lax.broadcasted_iota(jnp.int32, (_SC_LANES,), 0)
    out_scr[pl.ds(blk, _SC_LANES)] = jnp.where(idx == lane, thr_vec, old)


def _sc_thresh_body(d2_hbm, thr_hbm, buf0, buf1, out_scr, sem0, sem1):
    wid = lax.axis_index("s") * 2 + lax.axis_index("c")
    base = wid * _ROWS_PER_TILE
    last = base + _ROWS_PER_TILE - 1

    pltpu.async_copy(d2_hbm.at[base], buf0, sem0)
    pltpu.async_copy(d2_hbm.at[base + 1], buf1, sem1)

    def body(j, _):
        r0 = base + 2 * j
        pltpu.make_async_copy(d2_hbm.at[r0], buf0, sem0).wait()
        thr0 = _sc_row_threshold(buf0)

        @pl.when(r0 + 2 <= last)
        def _():
            pltpu.async_copy(d2_hbm.at[r0 + 2], buf0, sem0)

        _sc_store_lane(out_scr, 2 * j, thr0)

        pltpu.make_async_copy(d2_hbm.at[r0 + 1], buf1, sem1).wait()
        thr1 = _sc_row_threshold(buf1)

        @pl.when(r0 + 3 <= last)
        def _():
            pltpu.async_copy(d2_hbm.at[r0 + 3], buf1, sem1)

        _sc_store_lane(out_scr, 2 * j + 1, thr1)
        return 0

    lax.fori_loop(0, _ROWS_PER_TILE // 2, body, 0)
    pltpu.sync_copy(out_scr, thr_hbm.at[pl.ds(base, _ROWS_PER_TILE)])


def _sc_thresh(d2):
    mesh = plsc.VectorSubcoreMesh(core_axis_name="c", subcore_axis_name="s")
    thr = pl.kernel(
        _sc_thresh_body,
        out_type=jax.ShapeDtypeStruct((_N,), jnp.float32),
        mesh=mesh,
        compiler_params=pltpu.CompilerParams(needs_layout_passes=False),
        scratch_types=[
            pltpu.VMEM((_N,), jnp.float32),
            pltpu.VMEM((_N,), jnp.float32),
            pltpu.VMEM((_ROWS_PER_TILE,), jnp.float32),
            pltpu.SemaphoreType.DMA,
            pltpu.SemaphoreType.DMA,
        ],
    )(d2)
    return thr


def _iter_kernel(xn_ref, d2_ref, thr_ref, alpha_ref, y_out_ref,
                 y_scr, ynext_scr, sq_scr, yb_scr):
    alpha = alpha_ref[0, 0]
    y_scr[...] = xn_ref[...]

    for _t in range(_MAX_ITER):
        def sq_body(rb, _):
            yr = y_scr[pl.ds(rb * _BR, _BR), :]
            yb_scr[pl.ds(rb * _BR, _BR), :] = yr.astype(jnp.bfloat16)
            if _t > 0:
                sq_scr[0, pl.ds(rb * _BR, _BR)] = jnp.sum(yr * yr, axis=1)
            return 0

        lax.fori_loop(0, _NB, sq_body, 0)

        first = _t == 0

        def blk_body(rb, _):
            yr = y_scr[pl.ds(rb * _BR, _BR), :]
            d2r = d2_ref[pl.ds(rb * _BR, _BR), :]
            thr = thr_ref[pl.ds(rb * _BR, _BR), :]
            mask = d2r <= thr
            if first:
                rm = jnp.min(d2r, axis=1, keepdims=True)
                e = jnp.where(mask, jnp.exp(alpha * (rm - d2r)),
                              0.0).astype(jnp.bfloat16)
            else:
                yb = yb_scr[...]
                yrb = yb_scr[pl.ds(rb * _BR, _BR), :]
                g = lax.dot_general(yrb, yb, (((1,), (1,)), ((), ())),
                                    preferred_element_type=jnp.float32)
                sq_row = sq_scr[0, :][None, :]
                logits = alpha * (2.0 * g - sq_row)
                ml = jnp.where(mask, logits, -1e30)
                m = jnp.max(ml, axis=1, keepdims=True)
                e = jnp.where(mask, jnp.exp(ml - m), 0.0).astype(
                    jnp.bfloat16)
            s = jnp.sum(e.astype(jnp.float32), axis=1, keepdims=True)
            yfb = yb_scr[...]
            wy = lax.dot_general(e, yfb, (((1,), (0,)), ((), ())),
                                 preferred_element_type=jnp.float32)
            ynext_scr[pl.ds(rb * _BR, _BR), :] = yr + _ETA * (wy / s - yr)
            return 0

        lax.fori_loop(0, _NB, blk_body, 0)
        y_scr[...] = ynext_scr[...]

    y_out_ref[...] = y_scr[...]


def kernel(x, alpha):
    xn, d2 = pl.pallas_call(
        _prep_kernel,
        out_shape=[
            jax.ShapeDtypeStruct((_N, _D), jnp.float32),
            jax.ShapeDtypeStruct((_N, _N), jnp.float32),
        ],
    )(x)

    thr = _sc_thresh(d2).reshape(_N, 1)

    alpha2d = jnp.asarray(alpha, jnp.float32).reshape(1, 1)
    y = pl.pallas_call(
        _iter_kernel,
        in_specs=[
            pl.BlockSpec(memory_space=pltpu.VMEM),
            pl.BlockSpec(memory_space=pltpu.VMEM),
            pl.BlockSpec(memory_space=pltpu.VMEM),
            pl.BlockSpec(memory_space=pltpu.SMEM),
        ],
        out_shape=jax.ShapeDtypeStruct((_N, _D), jnp.float32),
        scratch_shapes=[
            pltpu.VMEM((_N, _D), jnp.float32),
            pltpu.VMEM((_N, _D), jnp.float32),
            pltpu.VMEM((1, _N), jnp.float32),
            pltpu.VMEM((_N, _D), jnp.bfloat16),
        ],
    )(xn, d2, thr, alpha2d)
    return y
